# baseline (speedup 1.0000x reference)
import functools

import jax
import jax.numpy as jnp
from jax import lax
from jax.experimental import pallas as pl
from jax.experimental.pallas import tpu as pltpu
from jax.experimental.pallas import tpu_sc as plsc

DIM = 32
NB = 16384
NS = 50
B = NB * NS
NCORE, NSUB = 2, 16
NW = NCORE * NSUB
BPW = NB // NW
KCH = 4
BC = BPW // KCH
SCH = 5
NU = NS // SCH
TRO = DIM // 8
TILE = 8 * BC
OUT_FLAT = NS * TRO * (NB // BC) * TILE


def _make_kernel():
    mesh = plsc.VectorSubcoreMesh(core_axis_name="c", subcore_axis_name="s")

    @functools.partial(
        pl.kernel,
        mesh=mesh,
        out_type=jax.ShapeDtypeStruct((OUT_FLAT,), jnp.float32),
        scratch_types=[
            pltpu.VMEM((BC * NS,), jnp.int32),
            pltpu.VMEM((NS, BC), jnp.int32),
            pltpu.VMEM((2, SCH * BC, DIM), jnp.float32),
            pltpu.VMEM((2, SCH, TRO * TILE), jnp.float32),
            pltpu.SemaphoreType.DMA,
            pltpu.SemaphoreType.DMA,
            pltpu.SemaphoreType.DMA,
            pltpu.SemaphoreType.DMA,
        ],
        compiler_params=pltpu.CompilerParams(
            use_tc_tiling_on_sc=False, needs_layout_passes=False
        ),
    )
    def gather_kernel(idx_hbm, table_hbm, out_hbm, ibuf, ibufT, gbuf, obuf,
                      g0, g1, o0, o1):
        gsem = (g0, g1)
        osem = (o0, o1)
        wid = lax.axis_index("s") * NCORE + lax.axis_index("c")
        iota16 = lax.iota(jnp.int32, 16)
        iota16x50 = iota16 * NS
        zeros16 = jnp.zeros((16,), jnp.int32)
        sc0_c = iota16 * BC
        sc1_c = sc0_c + 16 * BC

        def kbody(k, carry):
            base = (wid * BPW + k * BC) * NS
            tc = wid * KCH + k
            pltpu.sync_copy(idx_hbm.at[pl.ds(base, BC * NS)], ibuf)

            @plsc.parallel_loop(0, NS, unroll=2)
            def _idxt(s):
                for g in range(8):
                    rows = iota16x50 + (g * 16 * NS + s)
                    ibufT[s, pl.ds(g * 16, 16)] = plsc.load_gather(ibuf, [rows])

            def fire_g(u, b):
                for sl in range(SCH):
                    pltpu.async_copy(
                        table_hbm.at[ibufT.at[u * SCH + sl]],
                        gbuf.at[b, pl.ds(sl * BC, BC)],
                        gsem[b],
                    )

            def wait_g(b):
                pltpu.make_async_copy(
                    table_hbm.at[pl.ds(0, SCH * BC)], gbuf.at[b], gsem[b]
                ).wait()

            def fire_o(u, b):
                for sl in range(SCH):
                    for tr in range(TRO):
                        off = (((u * SCH + sl) * TRO + tr) * (NB // BC) + tc) * TILE
                        pltpu.async_copy(
                            obuf.at[b, sl, pl.ds(tr * TILE, TILE)],
                            out_hbm.at[pl.ds(off, TILE)],
                            osem[b],
                        )

            def wait_o(b):
                for sl in range(SCH):
                    pltpu.make_async_copy(
                        obuf.at[b, sl], out_hbm.at[pl.ds(0, TRO * TILE)], osem[b]
                    ).wait()

            def transpose(b):
                for s_l in range(SCH):
                    rbase = s_l * BC

                    @plsc.parallel_loop(0, BC, unroll=8)
                    def _row(bi):
                        v0 = gbuf[b, rbase + bi, pl.ds(0, 16)]
                        v1 = gbuf[b, rbase + bi, pl.ds(16, 16)]
                        bi_c = zeros16 + bi
                        plsc.store_scatter(obuf.at[b, s_l], [sc0_c + bi_c], v0)
                        plsc.store_scatter(obuf.at[b, s_l], [sc1_c + bi_c], v1)

            fire_g(0, 0)
            for u in range(NU):
                if u + 1 < NU:
                    fire_g(u + 1, (u + 1) % 2)
                wait_g(u % 2)
                if u >= 2:
                    wait_o(u % 2)
                transpose(u % 2)
                fire_o(u, u % 2)
            wait_o(0)
            wait_o(1)
            return carry

        lax.fori_loop(0, KCH, kbody, 0)

    return gather_kernel


_gather = _make_kernel()


def kernel(inputs, table):
    idx_flat = inputs.reshape(B).astype(jnp.int32)
    out_flat = _gather(idx_flat, table)
    out5 = out_flat.reshape(NS, TRO, NB // BC, 8, BC)
    t = out5.transpose(2, 4, 0, 1, 3)
    return t.reshape(NB, NS, DIM)


# speedup vs baseline: 1.5440x; 1.5440x over previous
import functools

import jax
import jax.numpy as jnp
from jax import lax
from jax.experimental import pallas as pl
from jax.experimental.pallas import tpu as pltpu
from jax.experimental.pallas import tpu_sc as plsc

DIM = 32
NB = 16384
NS = 50
B = NB * NS
NCORE, NSUB = 2, 16
NW = NCORE * NSUB
BPW = NB // NW
KCH = 4
BC = BPW // KCH
SCH = 5
NU = NS // SCH
TRO = DIM // 8
BCP = BC + 1
NTC = NB // BC


def _make_kernel():
    mesh = plsc.VectorSubcoreMesh(core_axis_name="c", subcore_axis_name="s")

    @functools.partial(
        pl.kernel,
        mesh=mesh,
        out_type=jax.ShapeDtypeStruct((NS * TRO * NTC, 8, BC), jnp.float32),
        scratch_types=[
            pltpu.VMEM((BC * NS,), jnp.int32),
            pltpu.VMEM((NS, BC), jnp.int32),
            pltpu.VMEM((2, SCH * BC, DIM), jnp.float32),
            pltpu.VMEM((2, SCH, DIM, BCP), jnp.float32),
            pltpu.SemaphoreType.DMA,
            pltpu.SemaphoreType.DMA,
            pltpu.SemaphoreType.DMA,
            pltpu.SemaphoreType.DMA,
        ],
        compiler_params=pltpu.CompilerParams(
            use_tc_tiling_on_sc=False, needs_layout_passes=False
        ),
    )
    def gather_kernel(idx_hbm, table_hbm, out_hbm, ibuf, ibufT, gbuf, obuf,
                      g0, g1, o0, o1):
        gsem = (g0, g1)
        osem = (o0, o1)
        wid = lax.axis_index("s") * NCORE + lax.axis_index("c")
        iota16 = lax.iota(jnp.int32, 16)
        iota16x50 = iota16 * NS
        zeros16 = jnp.zeros((16,), jnp.int32)
        d0_c = iota16
        d1_c = iota16 + 16

        def kbody(k, carry):
            base = (wid * BPW + k * BC) * NS
            tc = wid * KCH + k
            pltpu.sync_copy(idx_hbm.at[pl.ds(base, BC * NS)], ibuf)

            @plsc.parallel_loop(0, NS, unroll=2)
            def _idxt(s):
                for g in range(8):
                    rows = iota16x50 + (g * 16 * NS + s)
                    ibufT[s, pl.ds(g * 16, 16)] = plsc.load_gather(ibuf, [rows])

            def fire_g(u, b):
                for sl in range(SCH):
                    pltpu.async_copy(
                        table_hbm.at[ibufT.at[u * SCH + sl]],
                        gbuf.at[b, pl.ds(sl * BC, BC)],
                        gsem[b],
                    )

            def wait_g(b):
                pltpu.make_async_copy(
                    table_hbm.at[pl.ds(0, SCH * BC)], gbuf.at[b], gsem[b]
                ).wait()

            def fire_o(u, b):
                for sl in range(SCH):
                    for tr in range(TRO):
                        row = ((u * SCH + sl) * TRO + tr) * NTC + tc
                        pltpu.async_copy(
                            obuf.at[b, sl, pl.ds(tr * 8, 8), pl.ds(0, BC)],
                            out_hbm.at[row],
                            osem[b],
                        )

            def wait_o(b):
                for n in range(SCH * TRO):
                    pltpu.make_async_copy(
                        out_hbm.at[n], out_hbm.at[n], osem[b]
                    ).wait()

            def transpose(b):
                for s_l in range(SCH):
                    rbase = s_l * BC
                    oview = obuf.at[b, s_l]

                    @plsc.parallel_loop(0, BC, unroll=8)
                    def _row(bi):
                        v0 = gbuf[b, rbase + bi, pl.ds(0, 16)]
                        v1 = gbuf[b, rbase + bi, pl.ds(16, 16)]
                        bi_c = zeros16 + bi
                        plsc.store_scatter(oview, [d0_c, bi_c], v0)
                        plsc.store_scatter(oview, [d1_c, bi_c], v1)

            fire_g(0, 0)
            for u in range(NU):
                if u + 1 < NU:
                    fire_g(u + 1, (u + 1) % 2)
                wait_g(u % 2)
                if u >= 2:
                    wait_o(u % 2)
                transpose(u % 2)
                fire_o(u, u % 2)
            wait_o(0)
            wait_o(1)
            return carry

        lax.fori_loop(0, KCH, kbody, 0)

    return gather_kernel


_gather = _make_kernel()


def kernel(inputs, table):
    idx_flat = inputs.reshape(B).astype(jnp.int32)
    out3 = _gather(idx_flat, table)
    out5 = out3.reshape(NS, TRO, NTC, 8, BC)
    t = out5.transpose(2, 4, 0, 1, 3)
    return t.reshape(NB, NS, DIM)
